# initial kernel scaffold (unmeasured)
import functools

import jax
import jax.numpy as jnp
from jax import lax
from jax.experimental import pallas as pl
from jax.experimental.pallas import tpu as pltpu

N_DEV = 16
M, N = 4096, 8192
CH = M // N_DEV
N_HOPS = 2 * (N_DEV - 1)


def _gemm(x, w):
    m, k = x.shape
    _, n = w.shape
    bn = 1024

    def body(x_ref, w_ref, o_ref):
        o_ref[...] = jnp.dot(
            x_ref[...], w_ref[...], preferred_element_type=jnp.float32
        )

    return pl.pallas_call(
        body,
        grid=(n // bn,),
        in_specs=[
            pl.BlockSpec((m, k), lambda j: (0, 0)),
            pl.BlockSpec((k, bn), lambda j: (0, j)),
        ],
        out_specs=pl.BlockSpec((m, bn), lambda j: (0, j)),
        out_shape=jax.ShapeDtypeStruct((m, n), jnp.float32),
    )(x, w)


def _ar_body(
    part_ref,
    out_ref,
    acc,
    ptmp,
    own,
    slots,
    send_sems,
    recv_sems,
    load_sem,
    store_sems,
    own_sem,
    credit_sem,
):
    my = lax.axis_index("i")
    left = lax.rem(my - 1 + N_DEV, N_DEV)
    right = lax.rem(my + 1, N_DEV)

    barrier = pltpu.get_barrier_semaphore()
    for nbr in (left, right):
        pl.semaphore_signal(
            barrier, inc=1, device_id=(nbr,), device_id_type=pl.DeviceIdType.MESH
        )
    pl.semaphore_wait(barrier, 2)

    def send_credit_to_left():
        pl.semaphore_signal(
            credit_sem, inc=1, device_id=(left,), device_id_type=pl.DeviceIdType.MESH
        )

    cp = pltpu.make_async_copy(part_ref.at[pl.ds(my * CH, CH)], acc, load_sem)
    cp.start()
    cp.wait()

    for k in range(N_DEV - 1):
        slot = k % 2
        if k >= 2:
            pl.semaphore_wait(credit_sem, 1)
        rdma = pltpu.make_async_remote_copy(
            src_ref=acc,
            dst_ref=slots.at[slot],
            send_sem=send_sems.at[slot],
            recv_sem=recv_sems.at[slot],
            device_id=(right,),
            device_id_type=pl.DeviceIdType.MESH,
        )
        rdma.start()
        c = lax.rem(my - k - 1 + N_DEV, N_DEV)
        cp = pltpu.make_async_copy(part_ref.at[pl.ds(c * CH, CH)], ptmp, load_sem)
        cp.start()
        rdma.wait()
        cp.wait()
        if k < N_DEV - 2:
            acc[...] = slots[slot] + ptmp[...]
        else:
            own[...] = slots[slot] + ptmp[...]
        send_credit_to_left()

    oc = lax.rem(my + 1, N_DEV)
    cp_own = pltpu.make_async_copy(own, out_ref.at[pl.ds(oc * CH, CH)], own_sem)
    cp_own.start()

    prev_store = None
    for k in range(N_DEV - 1, N_HOPS):
        slot = k % 2
        pl.semaphore_wait(credit_sem, 1)
        src = own if k == N_DEV - 1 else slots.at[(k - 1) % 2]
        rdma = pltpu.make_async_remote_copy(
            src_ref=src,
            dst_ref=slots.at[slot],
            send_sem=send_sems.at[slot],
            recv_sem=recv_sems.at[slot],
            device_id=(right,),
            device_id_type=pl.DeviceIdType.MESH,
        )
        rdma.start()
        rdma.wait()
        if prev_store is not None:
            prev_store.wait()
            if k - 1 <= N_HOPS - 3:
                send_credit_to_left()
        t = k - (N_DEV - 1)
        r = lax.rem(my - t + N_DEV, N_DEV)
        st = pltpu.make_async_copy(
            slots.at[slot], out_ref.at[pl.ds(r * CH, CH)], store_sems.at[slot]
        )
        st.start()
        prev_store = st
    prev_store.wait()
    cp_own.wait()

    @functools.partial(pl.run_scoped, sem=pltpu.SemaphoreType.REGULAR)
    def _(sem):
        for nbr in (left, right):
            pl.semaphore_signal(
                sem, inc=1, device_id=(nbr,), device_id_type=pl.DeviceIdType.MESH
            )
        pl.semaphore_wait(sem, 2)


def _all_reduce(partial):
    return pl.pallas_call(
        _ar_body,
        out_shape=jax.ShapeDtypeStruct((M, N), jnp.float32),
        in_specs=[pl.BlockSpec(memory_space=pltpu.ANY)],
        out_specs=pl.BlockSpec(memory_space=pltpu.ANY),
        scratch_shapes=[
            pltpu.VMEM((CH, N), jnp.float32),
            pltpu.VMEM((CH, N), jnp.float32),
            pltpu.VMEM((CH, N), jnp.float32),
            pltpu.VMEM((2, CH, N), jnp.float32),
            pltpu.SemaphoreType.DMA((2,)),
            pltpu.SemaphoreType.DMA((2,)),
            pltpu.SemaphoreType.DMA,
            pltpu.SemaphoreType.DMA((2,)),
            pltpu.SemaphoreType.DMA,
            pltpu.SemaphoreType.REGULAR,
        ],
        compiler_params=pltpu.CompilerParams(collective_id=0),
    )(partial)


def kernel(x, w_mat):
    partial = _gemm(x, w_mat)
    y = _all_reduce(partial)
    amax = jnp.maximum(jnp.max(y), 0.0)
    scale = amax / 448.0
    q = (jnp.maximum(y, 0.0) / scale).astype(jnp.float8_e4m3fn)
    return q.astype(jnp.float32) * scale


# baseline (device time: 3080310 ns/iter reference)
import functools

import jax
import jax.numpy as jnp
from jax import lax
from jax.experimental import pallas as pl
from jax.experimental.pallas import tpu as pltpu

N_DEV = 16
M, N = 4096, 8192
CH = M // N_DEV
N_HOPS = 2 * (N_DEV - 1)


def _gemm(x, w):
    m, k = x.shape
    _, n = w.shape
    bn = 1024

    def body(x_ref, w_ref, o_ref):
        o_ref[...] = jnp.dot(
            x_ref[...],
            w_ref[...],
            preferred_element_type=jnp.float32,
            precision=lax.Precision.HIGHEST,
        )

    return pl.pallas_call(
        body,
        grid=(n // bn,),
        in_specs=[
            pl.BlockSpec((m, k), lambda j: (0, 0)),
            pl.BlockSpec((k, bn), lambda j: (0, j)),
        ],
        out_specs=pl.BlockSpec((m, bn), lambda j: (0, j)),
        out_shape=jax.ShapeDtypeStruct((m, n), jnp.float32),
        compiler_params=pltpu.CompilerParams(
            vmem_limit_bytes=100 * 1024 * 1024
        ),
    )(x, w)


def _ar_body(
    part_ref,
    out_ref,
    acc,
    ptmp,
    own,
    slots,
    send_sems,
    recv_sems,
    load_sem,
    store_sems,
    own_sem,
    credit_sem,
):
    my = lax.axis_index("i")
    left = lax.rem(my - 1 + N_DEV, N_DEV)
    right = lax.rem(my + 1, N_DEV)

    barrier = pltpu.get_barrier_semaphore()
    for nbr in (left, right):
        pl.semaphore_signal(
            barrier, inc=1, device_id=(nbr,), device_id_type=pl.DeviceIdType.MESH
        )
    pl.semaphore_wait(barrier, 2)

    def send_credit_to_left():
        pl.semaphore_signal(
            credit_sem, inc=1, device_id=(left,), device_id_type=pl.DeviceIdType.MESH
        )

    cp = pltpu.make_async_copy(part_ref.at[pl.ds(my * CH, CH)], acc, load_sem)
    cp.start()
    cp.wait()

    for k in range(N_DEV - 1):
        slot = k % 2
        if k >= 2:
            pl.semaphore_wait(credit_sem, 1)
        rdma = pltpu.make_async_remote_copy(
            src_ref=acc,
            dst_ref=slots.at[slot],
            send_sem=send_sems.at[slot],
            recv_sem=recv_sems.at[slot],
            device_id=(right,),
            device_id_type=pl.DeviceIdType.MESH,
        )
        rdma.start()
        c = lax.rem(my - k - 1 + N_DEV, N_DEV)
        cp = pltpu.make_async_copy(part_ref.at[pl.ds(c * CH, CH)], ptmp, load_sem)
        cp.start()
        rdma.wait()
        cp.wait()
        if k < N_DEV - 2:
            acc[...] = slots[slot] + ptmp[...]
        else:
            own[...] = slots[slot] + ptmp[...]
        send_credit_to_left()

    oc = lax.rem(my + 1, N_DEV)
    cp_own = pltpu.make_async_copy(own, out_ref.at[pl.ds(oc * CH, CH)], own_sem)
    cp_own.start()

    prev_store = None
    for k in range(N_DEV - 1, N_HOPS):
        slot = k % 2
        pl.semaphore_wait(credit_sem, 1)
        src = own if k == N_DEV - 1 else slots.at[(k - 1) % 2]
        rdma = pltpu.make_async_remote_copy(
            src_ref=src,
            dst_ref=slots.at[slot],
            send_sem=send_sems.at[slot],
            recv_sem=recv_sems.at[slot],
            device_id=(right,),
            device_id_type=pl.DeviceIdType.MESH,
        )
        rdma.start()
        rdma.wait()
        if prev_store is not None:
            prev_store.wait()
            if k - 1 <= N_HOPS - 3:
                send_credit_to_left()
        t = k - (N_DEV - 1)
        r = lax.rem(my - t + N_DEV, N_DEV)
        st = pltpu.make_async_copy(
            slots.at[slot], out_ref.at[pl.ds(r * CH, CH)], store_sems.at[slot]
        )
        st.start()
        prev_store = st
    prev_store.wait()
    cp_own.wait()

    @functools.partial(pl.run_scoped, sem=pltpu.SemaphoreType.REGULAR)
    def _(sem):
        for nbr in (left, right):
            pl.semaphore_signal(
                sem, inc=1, device_id=(nbr,), device_id_type=pl.DeviceIdType.MESH
            )
        pl.semaphore_wait(sem, 2)


def _all_reduce(partial):
    return pl.pallas_call(
        _ar_body,
        out_shape=jax.ShapeDtypeStruct((M, N), jnp.float32),
        in_specs=[pl.BlockSpec(memory_space=pl.ANY)],
        out_specs=pl.BlockSpec(memory_space=pl.ANY),
        scratch_shapes=[
            pltpu.VMEM((CH, N), jnp.float32),
            pltpu.VMEM((CH, N), jnp.float32),
            pltpu.VMEM((CH, N), jnp.float32),
            pltpu.VMEM((2, CH, N), jnp.float32),
            pltpu.SemaphoreType.DMA((2,)),
            pltpu.SemaphoreType.DMA((2,)),
            pltpu.SemaphoreType.DMA,
            pltpu.SemaphoreType.DMA((2,)),
            pltpu.SemaphoreType.DMA,
            pltpu.SemaphoreType.REGULAR,
        ],
        compiler_params=pltpu.CompilerParams(
            collective_id=0, vmem_limit_bytes=100 * 1024 * 1024
        ),
    )(partial)


def _snap_e4m3(a):
    bits = lax.bitcast_convert_type(a, jnp.uint32)
    p2 = lax.bitcast_convert_type(
        bits & jnp.uint32(0x7F800000), jnp.float32
    )
    q = jnp.where(a >= 2.0**-6, p2 * (2.0**-3), jnp.float32(2.0**-9))
    r = lax.round(a / q, lax.RoundingMethod.TO_NEAREST_EVEN) * q
    return jnp.minimum(r, 448.0)


def kernel(x, w_mat):
    partial = _gemm(x, w_mat)
    y = _all_reduce(partial)
    amax = jnp.maximum(jnp.max(y), 0.0)
    scale = amax / 448.0
    return _snap_e4m3(jnp.maximum(y, 0.0) / scale) * scale


# device time: 1973764 ns/iter; 1.5606x vs baseline; 1.5606x over previous
import functools

import jax
import jax.numpy as jnp
from jax import lax
from jax.experimental import pallas as pl
from jax.experimental.pallas import tpu as pltpu

N_DEV = 16
M, K, N = 4096, 256, 8192
CH = M // N_DEV

F32 = jnp.float32
F8 = jnp.float8_e4m3fn


def _snap_e4m3(a):
    bits = lax.bitcast_convert_type(a, jnp.uint32)
    qbits = lax.bitcast_convert_type(
        lax.bitcast_convert_type(bits & jnp.uint32(0x7F800000), F32) * (2.0**-3),
        jnp.uint32,
    )
    qbits = jnp.where(a >= 2.0**-6, qbits, jnp.uint32(0x3B000000))
    q = lax.bitcast_convert_type(qbits, F32)
    qinv = lax.bitcast_convert_type(jnp.uint32(254 << 23) - qbits, F32)
    v = a * qinv
    r = (v + 8388608.0) - 8388608.0
    return jnp.minimum(r * q, 448.0)


def _body(
    x_ref,
    w_ref,
    out_ref,
    acc,
    ptmp,
    slots_f,
    slots_q,
    maxbuf,
    send_sem_f,
    recv_sem_f,
    send_sems_q,
    recv_sems_q,
    store_sem,
    scal_send_sems,
    scal_recv_sems,
    credit_sem,
):
    my = lax.axis_index("i")
    left = lax.rem(my - 1 + N_DEV, N_DEV)
    right = lax.rem(my + 1, N_DEV)

    barrier = pltpu.get_barrier_semaphore()
    for nbr in (left, right):
        pl.semaphore_signal(
            barrier, inc=1, device_id=(nbr,), device_id_type=pl.DeviceIdType.MESH
        )
    pl.semaphore_wait(barrier, 2)

    def credit_to_left():
        pl.semaphore_signal(
            credit_sem, inc=1, device_id=(left,), device_id_type=pl.DeviceIdType.MESH
        )

    def chunk_gemm_into(dst, c):
        bn = 4096
        for j in range(N // bn):
            dst[:, j * bn : (j + 1) * bn] = jnp.dot(
                x_ref[pl.ds(c * CH, CH), :],
                w_ref[:, j * bn : (j + 1) * bn],
                preferred_element_type=F32,
                precision=lax.Precision.HIGHEST,
            )

    chunk_gemm_into(acc, my)

    def rs_hop(k, carry):
        @pl.when(k >= 1)
        def _():
            pl.semaphore_wait(credit_sem, 1)

        rdma = pltpu.make_async_remote_copy(
            src_ref=acc,
            dst_ref=slots_f,
            send_sem=send_sem_f,
            recv_sem=recv_sem_f,
            device_id=(right,),
            device_id_type=pl.DeviceIdType.MESH,
        )
        rdma.start()
        c = lax.rem(my - k - 1 + N_DEV, N_DEV)
        chunk_gemm_into(ptmp, c)
        rdma.wait()
        acc[...] = slots_f[...] + ptmp[...]

        @pl.when(k <= N_DEV - 3)
        def _():
            credit_to_left()

        return carry

    lax.fori_loop(0, N_DEV - 1, rs_hop, 0)

    m_local = jnp.max(acc[...])
    maxbuf[pl.ds(my, 1), :] = jnp.full((1, 128), m_local, F32)

    def scal_rdma(off, j):
        return pltpu.make_async_remote_copy(
            src_ref=maxbuf.at[pl.ds(my, 1)],
            dst_ref=maxbuf.at[pl.ds(j, 1)],
            send_sem=scal_send_sems.at[off - 1],
            recv_sem=scal_recv_sems.at[j],
            device_id=(lax.rem(my + off, N_DEV),),
            device_id_type=pl.DeviceIdType.MESH,
        )

    def sc_send(off, carry):
        scal_rdma(off, my).start()
        return carry

    def sc_wait(off, carry):
        scal_rdma(off, lax.rem(my + off, N_DEV)).wait_recv()
        scal_rdma(off, my).wait_send()
        return carry

    lax.fori_loop(1, N_DEV, sc_send, 0)
    lax.fori_loop(1, N_DEV, sc_wait, 0)

    amax = jnp.maximum(jnp.max(maxbuf[...]), 0.0)
    scale = amax / 448.0
    inv = jnp.where(amax > 0, 448.0 / amax, 0.0)

    ptmp[...] = _snap_e4m3(jnp.maximum(acc[...], 0.0) * inv)
    slots_q[0, :, :] = ptmp[...].astype(F8)
    ptmp[...] = ptmp[...] * scale
    oc = lax.rem(my + 1, N_DEV)
    cp_own = pltpu.make_async_copy(ptmp, out_ref.at[pl.ds(oc * CH, CH)], store_sem)
    cp_own.start()
    cp_own.wait()

    def ag_hop(t, carry):
        slot = lax.rem(t + 1, 2)
        oslot = 1 - slot

        @pl.when(t >= 1)
        def _():
            pl.semaphore_wait(credit_sem, 1)

        rdma = pltpu.make_async_remote_copy(
            src_ref=slots_q.at[oslot],
            dst_ref=slots_q.at[slot],
            send_sem=send_sems_q.at[slot],
            recv_sem=recv_sems_q.at[slot],
            device_id=(right,),
            device_id_type=pl.DeviceIdType.MESH,
        )
        rdma.start()
        rdma.wait()

        @pl.when(t >= 1)
        def _():
            prev_r = lax.rem(my - t + 1 + N_DEV, N_DEV)
            pltpu.make_async_copy(
                ptmp, out_ref.at[pl.ds(prev_r * CH, CH)], store_sem
            ).wait()

        @pl.when(t <= N_DEV - 3)
        def _():
            credit_to_left()

        r = lax.rem(my - t + N_DEV, N_DEV)
        ptmp[...] = slots_q[slot].astype(F32) * scale
        pltpu.make_async_copy(
            ptmp, out_ref.at[pl.ds(r * CH, CH)], store_sem
        ).start()
        return carry

    lax.fori_loop(0, N_DEV - 1, ag_hop, 0)
    r_last = lax.rem(my - (N_DEV - 2) + N_DEV, N_DEV)
    pltpu.make_async_copy(
        ptmp, out_ref.at[pl.ds(r_last * CH, CH)], store_sem
    ).wait()

    @functools.partial(pl.run_scoped, sem=pltpu.SemaphoreType.REGULAR)
    def _(sem):
        for nbr in (left, right):
            pl.semaphore_signal(
                sem, inc=1, device_id=(nbr,), device_id_type=pl.DeviceIdType.MESH
            )
        pl.semaphore_wait(sem, 2)


def kernel(x, w_mat):
    return pl.pallas_call(
        _body,
        out_shape=jax.ShapeDtypeStruct((M, N), F32),
        in_specs=[
            pl.BlockSpec(memory_space=pltpu.VMEM),
            pl.BlockSpec(memory_space=pltpu.VMEM),
        ],
        out_specs=pl.BlockSpec(memory_space=pl.ANY),
        scratch_shapes=[
            pltpu.VMEM((CH, N), F32),
            pltpu.VMEM((CH, N), F32),
            pltpu.VMEM((CH, N), F32),
            pltpu.VMEM((2, CH, N), F8),
            pltpu.VMEM((N_DEV, 128), F32),
            pltpu.SemaphoreType.DMA,
            pltpu.SemaphoreType.DMA,
            pltpu.SemaphoreType.DMA((2,)),
            pltpu.SemaphoreType.DMA((2,)),
            pltpu.SemaphoreType.DMA,
            pltpu.SemaphoreType.DMA((N_DEV - 1,)),
            pltpu.SemaphoreType.DMA((N_DEV,)),
            pltpu.SemaphoreType.REGULAR,
        ],
        compiler_params=pltpu.CompilerParams(
            collective_id=0, vmem_limit_bytes=60 * 1024 * 1024
        ),
    )(x, w_mat)


# device time: 1128033 ns/iter; 2.7307x vs baseline; 1.7497x over previous
import functools

import jax
import jax.numpy as jnp
from jax import lax
from jax.experimental import pallas as pl
from jax.experimental.pallas import tpu as pltpu

N_DEV = 16
M, K, N = 4096, 256, 8192
CH = M // N_DEV
NH = N // 2

F32 = jnp.float32
F8 = jnp.float8_e4m3fn


def _snap_e4m3(a):
    bits = lax.bitcast_convert_type(a, jnp.uint32)
    qbits = lax.bitcast_convert_type(
        lax.bitcast_convert_type(bits & jnp.uint32(0x7F800000), F32) * (2.0**-3),
        jnp.uint32,
    )
    qbits = jnp.where(a >= 2.0**-6, qbits, jnp.uint32(0x3B000000))
    q = lax.bitcast_convert_type(qbits, F32)
    qinv = lax.bitcast_convert_type(jnp.uint32(254 << 23) - qbits, F32)
    v = a * qinv
    r = (v + 8388608.0) - 8388608.0
    return jnp.minimum(r * q, 448.0)


def _body(
    x_ref,
    w_ref,
    out_ref,
    acc_p,
    acc_n,
    ptmp_p,
    ptmp_n,
    slot_f_p,
    slot_f_n,
    slots_q_p,
    slots_q_n,
    maxbuf,
    sems_f_p,
    sems_f_n,
    send_q_p,
    recv_q_p,
    send_q_n,
    recv_q_n,
    store_sem_p,
    store_sem_n,
    scal_send_sems,
    scal_recv_sems,
    credit_p,
    credit_n,
):
    my = lax.axis_index("i")
    left = lax.rem(my - 1 + N_DEV, N_DEV)
    right = lax.rem(my + 1, N_DEV)

    barrier = pltpu.get_barrier_semaphore()
    for nbr in (left, right):
        pl.semaphore_signal(
            barrier, inc=1, device_id=(nbr,), device_id_type=pl.DeviceIdType.MESH
        )
    pl.semaphore_wait(barrier, 2)

    def credit_to(sem, nbr):
        pl.semaphore_signal(
            sem, inc=1, device_id=(nbr,), device_id_type=pl.DeviceIdType.MESH
        )

    def gemm_half(dst, c, h):
        dst[...] = jnp.dot(
            x_ref[pl.ds(c * CH, CH), :],
            w_ref[:, h * NH : (h + 1) * NH],
            preferred_element_type=F32,
            precision=lax.Precision.HIGHEST,
        )

    gemm_half(acc_p, my, 0)
    gemm_half(acc_n, my, 1)

    def rs_hop(k, carry):
        @pl.when(k >= 1)
        def _():
            pl.semaphore_wait(credit_p, 1)
            pl.semaphore_wait(credit_n, 1)

        rdma_p = pltpu.make_async_remote_copy(
            src_ref=acc_p,
            dst_ref=slot_f_p,
            send_sem=sems_f_p.at[0],
            recv_sem=sems_f_p.at[1],
            device_id=(right,),
            device_id_type=pl.DeviceIdType.MESH,
        )
        rdma_n = pltpu.make_async_remote_copy(
            src_ref=acc_n,
            dst_ref=slot_f_n,
            send_sem=sems_f_n.at[0],
            recv_sem=sems_f_n.at[1],
            device_id=(left,),
            device_id_type=pl.DeviceIdType.MESH,
        )
        rdma_p.start()
        rdma_n.start()
        c_p = lax.rem(my - k - 1 + N_DEV, N_DEV)
        c_n = lax.rem(my + k + 1, N_DEV)
        gemm_half(ptmp_p, c_p, 0)
        gemm_half(ptmp_n, c_n, 1)
        rdma_p.wait()
        acc_p[...] = slot_f_p[...] + ptmp_p[...]
        rdma_n.wait()
        acc_n[...] = slot_f_n[...] + ptmp_n[...]

        @pl.when(k <= N_DEV - 3)
        def _():
            credit_to(credit_p, left)
            credit_to(credit_n, right)

        return carry

    lax.fori_loop(0, N_DEV - 1, rs_hop, 0)

    m_local = jnp.maximum(jnp.max(acc_p[...]), jnp.max(acc_n[...]))
    maxbuf[pl.ds(my, 1), :] = jnp.full((1, 128), m_local, F32)

    def scal_rdma(off, j):
        return pltpu.make_async_remote_copy(
            src_ref=maxbuf.at[pl.ds(my, 1)],
            dst_ref=maxbuf.at[pl.ds(j, 1)],
            send_sem=scal_send_sems.at[off - 1],
            recv_sem=scal_recv_sems.at[j],
            device_id=(lax.rem(my + off, N_DEV),),
            device_id_type=pl.DeviceIdType.MESH,
        )

    def sc_send(off, carry):
        scal_rdma(off, my).start()
        return carry

    def sc_wait(off, carry):
        scal_rdma(off, lax.rem(my + off, N_DEV)).wait_recv()
        scal_rdma(off, my).wait_send()
        return carry

    lax.fori_loop(1, N_DEV, sc_send, 0)
    lax.fori_loop(1, N_DEV, sc_wait, 0)

    amax = jnp.maximum(jnp.max(maxbuf[...]), 0.0)
    scale = amax / 448.0
    inv = jnp.where(amax > 0, 448.0 / amax, 0.0)

    ptmp_p[...] = _snap_e4m3(jnp.maximum(acc_p[...], 0.0) * inv)
    slots_q_p[0, :, :] = ptmp_p[...].astype(F8)
    ptmp_p[...] = ptmp_p[...] * scale
    oc_p = lax.rem(my + 1, N_DEV)
    cp_p = pltpu.make_async_copy(
        ptmp_p, out_ref.at[pl.ds(oc_p * CH, CH), pl.ds(0, NH)], store_sem_p
    )
    cp_p.start()
    ptmp_n[...] = _snap_e4m3(jnp.maximum(acc_n[...], 0.0) * inv)
    slots_q_n[0, :, :] = ptmp_n[...].astype(F8)
    ptmp_n[...] = ptmp_n[...] * scale
    oc_n = lax.rem(my - 1 + N_DEV, N_DEV)
    cp_n = pltpu.make_async_copy(
        ptmp_n, out_ref.at[pl.ds(oc_n * CH, CH), pl.ds(NH, NH)], store_sem_n
    )
    cp_n.start()
    cp_p.wait()
    cp_n.wait()

    def ag_hop(t, carry):
        slot = lax.rem(t + 1, 2)
        oslot = 1 - slot

        @pl.when(t >= 1)
        def _():
            pl.semaphore_wait(credit_p, 1)
            pl.semaphore_wait(credit_n, 1)

        rdma_p = pltpu.make_async_remote_copy(
            src_ref=slots_q_p.at[oslot],
            dst_ref=slots_q_p.at[slot],
            send_sem=send_q_p.at[slot],
            recv_sem=recv_q_p.at[slot],
            device_id=(right,),
            device_id_type=pl.DeviceIdType.MESH,
        )
        rdma_n = pltpu.make_async_remote_copy(
            src_ref=slots_q_n.at[oslot],
            dst_ref=slots_q_n.at[slot],
            send_sem=send_q_n.at[slot],
            recv_sem=recv_q_n.at[slot],
            device_id=(left,),
            device_id_type=pl.DeviceIdType.MESH,
        )
        rdma_p.start()
        rdma_n.start()
        rdma_p.wait()
        rdma_n.wait()

        @pl.when(t >= 1)
        def _():
            pr_p = lax.rem(my - t + 1 + N_DEV, N_DEV)
            pltpu.make_async_copy(
                ptmp_p, out_ref.at[pl.ds(pr_p * CH, CH), pl.ds(0, NH)], store_sem_p
            ).wait()
            pr_n = lax.rem(my + t - 1, N_DEV)
            pltpu.make_async_copy(
                ptmp_n, out_ref.at[pl.ds(pr_n * CH, CH), pl.ds(NH, NH)], store_sem_n
            ).wait()

        @pl.when(t <= N_DEV - 3)
        def _():
            credit_to(credit_p, left)
            credit_to(credit_n, right)

        r_p = lax.rem(my - t + N_DEV, N_DEV)
        ptmp_p[...] = slots_q_p[slot].astype(F32) * scale
        pltpu.make_async_copy(
            ptmp_p, out_ref.at[pl.ds(r_p * CH, CH), pl.ds(0, NH)], store_sem_p
        ).start()
        r_n = lax.rem(my + t, N_DEV)
        ptmp_n[...] = slots_q_n[slot].astype(F32) * scale
        pltpu.make_async_copy(
            ptmp_n, out_ref.at[pl.ds(r_n * CH, CH), pl.ds(NH, NH)], store_sem_n
        ).start()
        return carry

    lax.fori_loop(0, N_DEV - 1, ag_hop, 0)
    rl_p = lax.rem(my - (N_DEV - 2) + N_DEV, N_DEV)
    pltpu.make_async_copy(
        ptmp_p, out_ref.at[pl.ds(rl_p * CH, CH), pl.ds(0, NH)], store_sem_p
    ).wait()
    rl_n = lax.rem(my + N_DEV - 2, N_DEV)
    pltpu.make_async_copy(
        ptmp_n, out_ref.at[pl.ds(rl_n * CH, CH), pl.ds(NH, NH)], store_sem_n
    ).wait()

    @functools.partial(pl.run_scoped, sem=pltpu.SemaphoreType.REGULAR)
    def _(sem):
        for nbr in (left, right):
            pl.semaphore_signal(
                sem, inc=1, device_id=(nbr,), device_id_type=pl.DeviceIdType.MESH
            )
        pl.semaphore_wait(sem, 2)


def kernel(x, w_mat):
    return pl.pallas_call(
        _body,
        out_shape=jax.ShapeDtypeStruct((M, N), F32),
        in_specs=[
            pl.BlockSpec(memory_space=pltpu.VMEM),
            pl.BlockSpec(memory_space=pltpu.VMEM),
        ],
        out_specs=pl.BlockSpec(memory_space=pl.ANY),
        scratch_shapes=[
            pltpu.VMEM((CH, NH), F32),
            pltpu.VMEM((CH, NH), F32),
            pltpu.VMEM((CH, NH), F32),
            pltpu.VMEM((CH, NH), F32),
            pltpu.VMEM((CH, NH), F32),
            pltpu.VMEM((CH, NH), F32),
            pltpu.VMEM((2, CH, NH), F8),
            pltpu.VMEM((2, CH, NH), F8),
            pltpu.VMEM((N_DEV, 128), F32),
            pltpu.SemaphoreType.DMA((2,)),
            pltpu.SemaphoreType.DMA((2,)),
            pltpu.SemaphoreType.DMA((2,)),
            pltpu.SemaphoreType.DMA((2,)),
            pltpu.SemaphoreType.DMA((2,)),
            pltpu.SemaphoreType.DMA((2,)),
            pltpu.SemaphoreType.DMA,
            pltpu.SemaphoreType.DMA,
            pltpu.SemaphoreType.DMA((N_DEV - 1,)),
            pltpu.SemaphoreType.DMA((N_DEV,)),
            pltpu.SemaphoreType.REGULAR,
            pltpu.SemaphoreType.REGULAR,
        ],
        compiler_params=pltpu.CompilerParams(
            collective_id=0, vmem_limit_bytes=60 * 1024 * 1024
        ),
    )(x, w_mat)
